# initial kernel scaffold (unmeasured)
import jax
import jax.numpy as jnp
from jax import lax
from jax.experimental import pallas as pl
from jax.experimental.pallas import tpu as pltpu

N_DEV = 4
S_LOC = 2048
D = 1024
HQ = 8
DH = 128
SCALE = 0.08838834764831843


def _ring_attn_body(q_ref, k_ref, v_ref, out_ref, comm_ref, send_sems, recv_sems):
    my = lax.axis_index("i")
    left = (my - 1) % N_DEV
    right = (my + 1) % N_DEV

    barrier_sem = pltpu.get_barrier_semaphore()
    for nbr in [left, right]:
        pl.semaphore_signal(
            barrier_sem, inc=1,
            device_id=(nbr,), device_id_type=pl.DeviceIdType.MESH,
        )
    pl.semaphore_wait(barrier_sem, 2)

    comm_ref[0, 0] = k_ref[...]
    comm_ref[0, 1] = v_ref[...]

    m = [None] * HQ
    l = [None] * HQ
    acc = [None] * HQ

    for hop in range(N_DEV):
        if hop < N_DEV - 1:
            rdma = pltpu.make_async_remote_copy(
                src_ref=comm_ref.at[hop],
                dst_ref=comm_ref.at[hop + 1],
                send_sem=send_sems.at[hop],
                recv_sem=recv_sems.at[hop],
                device_id=(right,),
                device_id_type=pl.DeviceIdType.MESH,
            )
            rdma.start()

        for h in range(HQ):
            q_h = q_ref[:, h * DH:(h + 1) * DH]
            k_h = comm_ref[hop, 0, :, h * DH:(h + 1) * DH]
            v_h = comm_ref[hop, 1, :, h * DH:(h + 1) * DH]
            s = lax.dot_general(
                q_h, k_h, (((1,), (1,)), ((), ())),
                preferred_element_type=jnp.float32,
            ) * SCALE
            if hop == 0:
                m_new = jnp.max(s, axis=1)
                p = jnp.exp(s - m_new[:, None])
                l[h] = jnp.sum(p, axis=1)
                acc[h] = lax.dot_general(
                    p.astype(jnp.bfloat16), v_h, (((1,), (0,)), ((), ())),
                    preferred_element_type=jnp.float32,
                )
            else:
                m_new = jnp.maximum(m[h], jnp.max(s, axis=1))
                alpha = jnp.exp(m[h] - m_new)
                p = jnp.exp(s - m_new[:, None])
                l[h] = l[h] * alpha + jnp.sum(p, axis=1)
                acc[h] = acc[h] * alpha[:, None] + lax.dot_general(
                    p.astype(jnp.bfloat16), v_h, (((1,), (0,)), ((), ())),
                    preferred_element_type=jnp.float32,
                )
            m[h] = m_new

        if hop < N_DEV - 1:
            rdma.wait()

    for h in range(HQ):
        out_ref[:, h * DH:(h + 1) * DH] = acc[h] / l[h][:, None]


def _ring_attn(q, k, v):
    return pl.pallas_call(
        _ring_attn_body,
        out_shape=jax.ShapeDtypeStruct((S_LOC, D), jnp.float32),
        in_specs=[pl.BlockSpec(memory_space=pltpu.VMEM)] * 3,
        out_specs=pl.BlockSpec(memory_space=pltpu.VMEM),
        scratch_shapes=[
            pltpu.VMEM((N_DEV, 2, S_LOC, D), jnp.bfloat16),
            pltpu.SemaphoreType.DMA((N_DEV - 1,)),
            pltpu.SemaphoreType.DMA((N_DEV - 1,)),
        ],
        compiler_params=pltpu.CompilerParams(collective_id=0),
    )(q, k, v)


def kernel(x, Wq, Wk, Wv, Wo):
    xb = x[0].astype(jnp.bfloat16)
    q = jnp.dot(xb, Wq.astype(jnp.bfloat16), preferred_element_type=jnp.float32)
    k = jnp.dot(xb, Wk.astype(jnp.bfloat16), preferred_element_type=jnp.float32)
    v = jnp.dot(xb, Wv.astype(jnp.bfloat16), preferred_element_type=jnp.bfloat16)

    my = lax.axis_index("i")
    pos = (my * S_LOC + jnp.arange(S_LOC)).astype(jnp.float32)
    inv = 1.0 / (10000.0 ** (jnp.arange(0, DH, 2, dtype=jnp.float32) / DH))
    ang = pos[:, None] * inv[None, :]
    cos = jnp.repeat(jnp.cos(ang), 2, axis=-1)
    sin = jnp.repeat(jnp.sin(ang), 2, axis=-1)

    def rope(t):
        t4 = t.reshape(S_LOC, HQ, DH // 2, 2)
        t_r = jnp.stack([-t4[..., 1], t4[..., 0]], axis=-1).reshape(S_LOC, HQ, DH)
        th = t.reshape(S_LOC, HQ, DH)
        return (th * cos[:, None, :] + t_r * sin[:, None, :]).reshape(S_LOC, D)

    qb = rope(q).astype(jnp.bfloat16)
    kb = rope(k).astype(jnp.bfloat16)

    ctx = _ring_attn(qb, kb, v)

    out = jnp.dot(ctx.astype(jnp.bfloat16), Wo.astype(jnp.bfloat16),
                  preferred_element_type=jnp.float32)
    return out[None]


# baseline (device time: 507010 ns/iter reference)
import jax
import jax.numpy as jnp
from jax import lax
from jax.experimental import pallas as pl
from jax.experimental.pallas import tpu as pltpu

N_DEV = 4
S_LOC = 2048
D = 1024
HQ = 8
DH = 128
QB = 256
NQB = S_LOC // QB
SCALE = 0.08838834764831843


def _ring_attn_body(q_ref, kv_ref, out_ref, comm_ref, acc_ref, ml_ref,
                    send_sems, recv_sems, credit_sem):
    my = lax.axis_index("i")
    left = (my - 1) % N_DEV
    right = (my + 1) % N_DEV

    barrier_sem = pltpu.get_barrier_semaphore()
    for nbr in [left, right]:
        pl.semaphore_signal(
            barrier_sem, inc=1,
            device_id=(nbr,), device_id_type=pl.DeviceIdType.MESH,
        )
    pl.semaphore_wait(barrier_sem, 2)

    for hop in range(N_DEV):
        if hop < N_DEV - 1:
            if hop == 2:
                pl.semaphore_wait(credit_sem, 1)
            rdma = pltpu.make_async_remote_copy(
                src_ref=kv_ref if hop == 0 else comm_ref.at[(hop - 1) % 2],
                dst_ref=comm_ref.at[hop % 2],
                send_sem=send_sems.at[hop],
                recv_sem=recv_sems.at[hop],
                device_id=(right,),
                device_id_type=pl.DeviceIdType.MESH,
            )
            rdma.start()

        k_src = kv_ref.at[0] if hop == 0 else comm_ref.at[(hop - 1) % 2, 0]
        v_src = kv_ref.at[1] if hop == 0 else comm_ref.at[(hop - 1) % 2, 1]
        first = hop == 0

        def head_body(h, _, k_src=k_src, v_src=v_src, first=first):
            hds = pl.ds(h * DH, DH)
            k_h = k_src[:, hds]
            v_h = v_src[:, hds]

            def qb_body(qb, _):
                qsl = pl.ds(qb * QB, QB)
                q_blk = q_ref[qsl, hds]
                s_t = lax.dot_general(
                    k_h, q_blk, (((1,), (1,)), ((), ())),
                    preferred_element_type=jnp.float32,
                )
                if first:
                    m_new = jnp.max(s_t, axis=0, keepdims=True)
                    p = jnp.exp(s_t - m_new)
                    l_new = jnp.sum(p, axis=0, keepdims=True)
                    acc_new = lax.dot_general(
                        v_h, p.astype(jnp.bfloat16),
                        (((0,), (0,)), ((), ())),
                        preferred_element_type=jnp.float32,
                    )
                else:
                    m_old = ml_ref[h, 0:1, qsl]
                    l_old = ml_ref[h, 1:2, qsl]
                    m_new = jnp.maximum(
                        m_old, jnp.max(s_t, axis=0, keepdims=True))
                    alpha = jnp.exp(m_old - m_new)
                    p = jnp.exp(s_t - m_new)
                    l_new = l_old * alpha + jnp.sum(p, axis=0, keepdims=True)
                    acc_new = acc_ref[h, :, qsl] * alpha + lax.dot_general(
                        v_h, p.astype(jnp.bfloat16),
                        (((0,), (0,)), ((), ())),
                        preferred_element_type=jnp.float32,
                    )
                ml_ref[h, 0:1, qsl] = m_new
                ml_ref[h, 1:2, qsl] = l_new
                acc_ref[h, :, qsl] = acc_new
                return 0

            lax.fori_loop(0, NQB, qb_body, 0)
            return 0

        lax.fori_loop(0, HQ, head_body, 0)

        if hop == 1:
            pl.semaphore_signal(
                credit_sem, inc=1,
                device_id=(left,), device_id_type=pl.DeviceIdType.MESH,
            )
        if hop < N_DEV - 1:
            rdma.wait()

    eye = (lax.broadcasted_iota(jnp.int32, (DH, DH), 0)
           == lax.broadcasted_iota(jnp.int32, (DH, DH), 1)).astype(jnp.float32)
    for h in range(HQ):
        ctx_t = acc_ref[h] / ml_ref[h, 1:2, :]
        out_blk = lax.dot_general(
            ctx_t, eye, (((0,), (0,)), ((), ())),
            preferred_element_type=jnp.float32,
        )
        out_ref[:, h * DH:(h + 1) * DH] = out_blk.astype(jnp.bfloat16)


def _ring_attn(q, kv):
    return pl.pallas_call(
        _ring_attn_body,
        out_shape=jax.ShapeDtypeStruct((S_LOC, D), jnp.bfloat16),
        in_specs=[pl.BlockSpec(memory_space=pltpu.VMEM)] * 2,
        out_specs=pl.BlockSpec(memory_space=pltpu.VMEM),
        scratch_shapes=[
            pltpu.VMEM((2, 2, S_LOC, D), jnp.bfloat16),
            pltpu.VMEM((HQ, DH, S_LOC), jnp.float32),
            pltpu.VMEM((HQ, 2, S_LOC), jnp.float32),
            pltpu.SemaphoreType.DMA((N_DEV - 1,)),
            pltpu.SemaphoreType.DMA((N_DEV - 1,)),
            pltpu.SemaphoreType.REGULAR,
        ],
        compiler_params=pltpu.CompilerParams(collective_id=0),
    )(q, kv)


def kernel(x, Wq, Wk, Wv, Wo):
    xb = x[0].astype(jnp.bfloat16)
    q = jnp.dot(xb, Wq.astype(jnp.bfloat16), preferred_element_type=jnp.float32)
    k = jnp.dot(xb, Wk.astype(jnp.bfloat16), preferred_element_type=jnp.float32)
    v = jnp.dot(xb, Wv.astype(jnp.bfloat16), preferred_element_type=jnp.bfloat16)

    my = lax.axis_index("i")
    pos = (my * S_LOC + jnp.arange(S_LOC)).astype(jnp.float32)
    inv = 1.0 / (10000.0 ** (jnp.arange(0, DH, 2, dtype=jnp.float32) / DH))
    ang = pos[:, None] * inv[None, :]
    cos = jnp.repeat(jnp.cos(ang), 2, axis=-1)
    sin = jnp.repeat(jnp.sin(ang), 2, axis=-1)

    def rope(t):
        t4 = t.reshape(S_LOC, HQ, DH // 2, 2)
        t_r = jnp.stack([-t4[..., 1], t4[..., 0]], axis=-1).reshape(S_LOC, HQ, DH)
        th = t.reshape(S_LOC, HQ, DH)
        return (th * cos[:, None, :] + t_r * sin[:, None, :]).reshape(S_LOC, D)

    qb = (rope(q) * SCALE).astype(jnp.bfloat16)
    kv = jnp.stack([rope(k).astype(jnp.bfloat16), v])

    ctx = _ring_attn(qb, kv)

    out = jnp.dot(ctx, Wo.astype(jnp.bfloat16),
                  preferred_element_type=jnp.float32)
    return out[None]
